# baseline (speedup 1.0000x reference)
import functools

import jax
import jax.numpy as jnp
from jax import lax
from jax.experimental import pallas as pl
from jax.experimental.pallas import tpu as pltpu
from jax.experimental.pallas import tpu_sc as plsc

N_NODES = 10000
N_NODES_PAD = 10112
N_EDGES = 8192
HE_NNZ = 32768
FEAT = 128
MAP = 64
LOG2 = 0.6931471805599453

NC = 2
NS = 16


def _leaky(x):
    return jnp.where(x >= 0, x, 0.01 * x)


def _sc_mesh():
    return plsc.VectorSubcoreMesh(core_axis_name="c", subcore_axis_name="s")


def _zero_table(zeros_hbm, table, sid, rows_per_tile):
    r0 = sid * rows_per_tile
    pltpu.sync_copy(zeros_hbm.at[pl.ds(r0, rows_per_tile)],
                    table.at[pl.ds(r0, rows_per_tile)])


def _writeback(table, out_hbm, sid, rows_per_tile):
    r0 = sid * rows_per_tile
    pltpu.sync_copy(table.at[pl.ds(r0, rows_per_tile)],
                    out_hbm.at[pl.ds(r0, rows_per_tile)])


def _ones_scatter_tile(idx_hbm, table, idx_v, ones_v, sid, n_items):
    per_tile = n_items // NS
    for c in range(per_tile // 256):
        off = sid * per_tile + c * 256
        pltpu.sync_copy(idx_hbm.at[pl.ds(off, 256)], idx_v)
        pltpu.sync_copy(ones_v, table.at[idx_v], add=True)


def _hist_body(col_hbm, node_hbm, he_hbm, ones_hbm, zeros_hbm,
               hc_out, hd_out, hb_out,
               table, idx_v, ones_v):
    cid = lax.axis_index("c")
    sid = lax.axis_index("s")
    pltpu.sync_copy(ones_hbm, ones_v)

    @pl.when(cid == 0)
    def _core0():
        _zero_table(zeros_hbm, table, sid, N_NODES_PAD // NS)
        plsc.subcore_barrier()
        _ones_scatter_tile(col_hbm, table, idx_v, ones_v, sid, N_EDGES)
        plsc.subcore_barrier()
        _writeback(table, hc_out, sid, N_NODES_PAD // NS)
        plsc.subcore_barrier()
        _zero_table(zeros_hbm, table, sid, N_EDGES // NS)
        plsc.subcore_barrier()
        _ones_scatter_tile(node_hbm, table, idx_v, ones_v, sid, HE_NNZ)
        plsc.subcore_barrier()
        _writeback(table, hd_out, sid, N_EDGES // NS)

    @pl.when(cid == 1)
    def _core1():
        _zero_table(zeros_hbm, table, sid, N_EDGES // NS)
        plsc.subcore_barrier()
        _ones_scatter_tile(he_hbm, table, idx_v, ones_v, sid, HE_NNZ)
        plsc.subcore_barrier()
        _writeback(table, hb_out, sid, N_EDGES // NS)


def _sc_hist(col_idx, node_idx, he_idx, ones128, zeros128):
    f32 = jnp.float32
    fn = pl.kernel(
        _hist_body,
        mesh=_sc_mesh(),
        out_type=[
            jax.ShapeDtypeStruct((N_NODES_PAD, FEAT), f32),
            jax.ShapeDtypeStruct((N_EDGES, FEAT), f32),
            jax.ShapeDtypeStruct((N_EDGES, FEAT), f32),
        ],
        scratch_types=[
            pltpu.VMEM_SHARED((N_NODES_PAD, FEAT), f32),
            pltpu.VMEM((256,), jnp.int32),
            pltpu.VMEM((256, FEAT), f32),
        ],
    )
    return fn(col_idx, node_idx, he_idx, ones128, zeros128)


def _partial_body(src_hbm, gidx_hbm, sidx_hbm, zeros_hbm, acc_out,
                  table, idx_v, rows_v, sem, *, n_items, n_rows):
    cid = lax.axis_index("c")
    sid = lax.axis_index("s")
    wid = sid * NC + cid
    rpt = n_rows // NS
    _zero_table(zeros_hbm, table, sid, rpt)
    plsc.subcore_barrier()
    per_w = n_items // (NC * NS)
    for c in range(per_w // 256):
        off = wid * per_w + c * 256
        pltpu.sync_copy(gidx_hbm.at[pl.ds(off, 256)], idx_v)
        pltpu.async_copy(src_hbm.at[idx_v], rows_v, sem).wait()
        pltpu.sync_copy(sidx_hbm.at[pl.ds(off, 256)], idx_v)
        pltpu.sync_copy(rows_v, table.at[idx_v], add=True)
    plsc.subcore_barrier()
    r0 = sid * rpt
    pltpu.sync_copy(table.at[pl.ds(r0, rpt)], acc_out.at[cid, pl.ds(r0, rpt)])


def _sc_scatter(src, gidx, sidx, zeros128, n_items, n_rows):
    f32 = jnp.float32
    body = functools.partial(_partial_body, n_items=n_items, n_rows=n_rows)
    fn = pl.kernel(
        body,
        mesh=_sc_mesh(),
        out_type=jax.ShapeDtypeStruct((NC, n_rows, FEAT), f32),
        scratch_types=[
            pltpu.VMEM_SHARED((n_rows, FEAT), f32),
            pltpu.VMEM((256,), jnp.int32),
            pltpu.VMEM((256, FEAT), f32),
            pltpu.SemaphoreType.DMA,
        ],
    )
    return fn(src, gidx, sidx, zeros128)


def _pq_body(p_hbm, q_hbm, row_hbm, col_hbm, pg_out, qg_out,
             idx_v, rows_v, sem):
    cid = lax.axis_index("c")
    sid = lax.axis_index("s")

    @pl.when(cid == 0)
    def _core0():
        for c in range(2):
            base = sid * 512 + c * 256
            pltpu.sync_copy(row_hbm.at[pl.ds(base, 256)], idx_v)
            pltpu.async_copy(p_hbm.at[idx_v], rows_v, sem).wait()
            pltpu.sync_copy(rows_v, pg_out.at[pl.ds(base, 256)])

    @pl.when(cid == 1)
    def _core1():
        for c in range(2):
            base = sid * 512 + c * 256
            pltpu.sync_copy(col_hbm.at[pl.ds(base, 256)], idx_v)
            pltpu.async_copy(q_hbm.at[idx_v], rows_v, sem).wait()
            pltpu.sync_copy(rows_v, qg_out.at[pl.ds(base, 256)])


def _sc_pq(p, q, row_idx, col_idx):
    f32 = jnp.float32
    fn = pl.kernel(
        _pq_body,
        mesh=_sc_mesh(),
        out_type=[
            jax.ShapeDtypeStruct((N_EDGES, FEAT), f32),
            jax.ShapeDtypeStruct((N_EDGES, FEAT), f32),
        ],
        scratch_types=[
            pltpu.VMEM((256,), jnp.int32),
            pltpu.VMEM((256, FEAT), f32),
            pltpu.SemaphoreType.DMA,
        ],
    )
    return fn(p, q, row_idx, col_idx)


def _dinv_from_hist(hc_ref):
    h = hc_ref[:, 0] + 1.0
    return (1.0 / jnp.sqrt(h))[:N_NODES, None]


def _recip_from_hist(hr_ref):
    h = hr_ref[:, 0]
    return jnp.where(h > 0, 1.0 / h, 0.0)[:, None]


def _mm1_body(nodes_ref, w1_ref, edges_ref, wh_ref, xw_ref, g_ref):
    xw_ref[...] = jnp.dot(nodes_ref[...], w1_ref[...],
                          preferred_element_type=jnp.float32)
    g_ref[...] = jnp.dot(edges_ref[...], wh_ref[...],
                         preferred_element_type=jnp.float32)


def _tc_mm1(nodes, w1, edges, wh):
    return pl.pallas_call(
        _mm1_body,
        out_shape=[jax.ShapeDtypeStruct((N_NODES, FEAT), jnp.float32),
                   jax.ShapeDtypeStruct((N_EDGES, FEAT), jnp.float32)],
    )(nodes, w1, edges, wh)


def _scale_body(hc_ref, xw_ref, xs_ref):
    xs_ref[...] = _dinv_from_hist(hc_ref) * xw_ref[...]


def _tc_scale(hc, xw):
    return pl.pallas_call(
        _scale_body,
        out_shape=jax.ShapeDtypeStruct((N_NODES, FEAT), jnp.float32),
    )(hc, xw)


def _gcnfin_body(hc_ref, accg_ref, xs_ref, b_ref, w_ref, out_ref):
    dinv = _dinv_from_hist(hc_ref)
    acc = accg_ref[0, :N_NODES, :] + accg_ref[1, :N_NODES, :]
    h = _leaky(dinv * (acc + xs_ref[...]) + b_ref[...])
    xw = jnp.dot(h, w_ref[...], preferred_element_type=jnp.float32)
    out_ref[...] = dinv * xw


def _tc_gcnfin(hc, accg, xs, b, w):
    return pl.pallas_call(
        _gcnfin_body,
        out_shape=jax.ShapeDtypeStruct((N_NODES, FEAT), jnp.float32),
    )(hc, accg, xs, b[None, :], w)


def _gcnfin2_body(hc_ref, accg_ref, xs2_ref, b2_ref, nwa_ref, nwb_ref,
                  p_ref, q_ref):
    dinv = _dinv_from_hist(hc_ref)
    acc = accg_ref[0, :N_NODES, :] + accg_ref[1, :N_NODES, :]
    ne = _leaky(dinv * (acc + xs2_ref[...]) + b2_ref[...])
    p_ref[...] = jnp.dot(ne, nwa_ref[...], preferred_element_type=jnp.float32)
    q_ref[...] = jnp.dot(ne, nwb_ref[...], preferred_element_type=jnp.float32)


def _tc_gcnfin2(hc, accg, xs2, b2, nwa, nwb):
    return pl.pallas_call(
        _gcnfin2_body,
        out_shape=[jax.ShapeDtypeStruct((N_NODES, FEAT), jnp.float32),
                   jax.ShapeDtypeStruct((N_NODES, FEAT), jnp.float32)],
    )(hc, accg, xs2, b2[None, :], nwa, nwb)


def _hypfin_body(hd_ref, acch_ref, hb1_ref, hw2_ref, gw2_ref):
    dinv_h = _recip_from_hist(hd_ref)
    g2 = _leaky(dinv_h * (acch_ref[0] + acch_ref[1]) + hb1_ref[...])
    gw2_ref[...] = jnp.dot(g2, hw2_ref[...], preferred_element_type=jnp.float32)


def _tc_hypfin(hd, acch, hb1, hw2):
    return pl.pallas_call(
        _hypfin_body,
        out_shape=jax.ShapeDtypeStruct((N_EDGES, FEAT), jnp.float32),
    )(hd, acch, hb1[None, :], hw2)


def _he_body(hb_ref, acch_ref, he_ref):
    he_ref[...] = _recip_from_hist(hb_ref) * (acch_ref[0] + acch_ref[1])


def _tc_hescale(hb, acch):
    return pl.pallas_call(
        _he_body,
        out_shape=jax.ShapeDtypeStruct((N_EDGES, FEAT), jnp.float32),
    )(hb, acch)


def _maps_body(hd_ref, acch_ref, hb2_ref, ew_ref, eb_ref,
               pg_ref, qg_ref, nb_ref, nm_ref, em_ref):
    dinv_h = _recip_from_hist(hd_ref)
    ee = _leaky(dinv_h * (acch_ref[0] + acch_ref[1]) + hb2_ref[...])
    emap = jnp.dot(ee, ew_ref[...], preferred_element_type=jnp.float32)
    emap = emap + eb_ref[...]
    nmap = (pg_ref[...] + qg_ref[...])[:, :MAP] + nb_ref[...]
    nm_ref[...] = nmap * lax.rsqrt(jnp.sum(nmap * nmap, axis=1,
                                           keepdims=True))
    em_ref[...] = emap * lax.rsqrt(jnp.sum(emap * emap, axis=1,
                                           keepdims=True))


def _tc_maps(hd, acch, hb2, ew, eb, pg, qg, nb):
    return pl.pallas_call(
        _maps_body,
        out_shape=[jax.ShapeDtypeStruct((N_EDGES, MAP), jnp.float32),
                   jax.ShapeDtypeStruct((N_EDGES, MAP), jnp.float32)],
    )(hd, acch, hb2[None, :], ew, eb[None, :], pg, qg, nb[None, :])


def _contrast_body(nm_ref, em_ref, out_ref, rs_ref, cs_ref, d_ref, *, bj, e):
    j = pl.program_id(0)
    nj = pl.num_programs(0)
    nm = nm_ref[...]
    em = em_ref[...]
    s = lax.dot_general(nm, em, (((1,), (1,)), ((), ())),
                        preferred_element_type=jnp.float32)
    z = jnp.exp(-jnp.abs(s))

    zparts = [z[:, k * 128:(k + 1) * 128] for k in range(bj // 128)]
    acc = zparts[0]
    for zp in zparts[1:]:
        acc = acc + zp

    @pl.when(j == 0)
    def _init():
        rs_ref[...] = acc

    @pl.when(j > 0)
    def _acc():
        rs_ref[...] += acc

    cs_ref[0, pl.ds(j * bj, bj)] = jnp.sum(z, axis=0)

    nm_blk = nm_ref[pl.ds(j * bj, bj), :]
    d_ref[0, pl.ds(j * bj, bj)] = jnp.sum(nm_blk * em, axis=1)

    @pl.when(j == nj - 1)
    def _fin():
        rowsum = jnp.sum(rs_ref[...], axis=1)
        out_ref[0, :] = (jnp.abs(d_ref[0, :]) - LOG2
                         + jnp.log(rowsum + cs_ref[0, :]))


def _contrast(nm, em, *, bj=512, interpret=False):
    e = nm.shape[0]
    nj = e // bj
    body = functools.partial(_contrast_body, bj=bj, e=e)
    out = pl.pallas_call(
        body,
        grid=(nj,),
        in_specs=[
            pl.BlockSpec((e, nm.shape[1]), lambda j: (0, 0)),
            pl.BlockSpec((bj, em.shape[1]), lambda j: (j, 0)),
        ],
        out_specs=pl.BlockSpec((1, e), lambda j: (0, 0)),
        out_shape=jax.ShapeDtypeStruct((1, e), jnp.float32),
        scratch_shapes=[
            pltpu.VMEM((e, 128), jnp.float32),
            pltpu.VMEM((1, e), jnp.float32),
            pltpu.VMEM((1, e), jnp.float32),
        ],
        interpret=interpret,
    )(nm, em)
    return out[0]


def kernel(nodes_feature, edges_feature, edge_index, hyperedge_index,
           gcn_w1, gcn_b1, gcn_w2, gcn_b2,
           hgc_w1, hgc_b1, hgc_w2, hgc_b2,
           node_w, node_b, edge_w, edge_b):
    f32 = jnp.float32
    row_idx = edge_index[0]
    col_idx = edge_index[1]
    node_idx = hyperedge_index[0]
    he_idx = hyperedge_index[1]

    ones128 = jnp.ones((256, FEAT), f32)
    zeros128 = jnp.zeros((N_NODES_PAD, FEAT), f32)

    hc, hd, hb = _sc_hist(col_idx, node_idx, he_idx, ones128, zeros128)
    xw1, g1 = _tc_mm1(nodes_feature, gcn_w1, edges_feature, hgc_w1)
    xs1 = _tc_scale(hc, xw1)

    acch1a = _sc_scatter(g1, node_idx, he_idx, zeros128, HE_NNZ, N_EDGES)
    accg1 = _sc_scatter(xs1, row_idx, col_idx, zeros128, N_EDGES, N_NODES_PAD)
    he1 = _tc_hescale(hb, acch1a)
    xs2 = _tc_gcnfin(hc, accg1, xs1, gcn_b1, gcn_w2)

    acch1b = _sc_scatter(he1, he_idx, node_idx, zeros128, HE_NNZ, N_EDGES)
    accg2 = _sc_scatter(xs2, row_idx, col_idx, zeros128, N_EDGES, N_NODES_PAD)
    gw2 = _tc_hypfin(hd, acch1b, hgc_b1, hgc_w2)
    wpad = jnp.zeros((FEAT, FEAT - MAP), f32)
    nwa = jnp.concatenate([node_w[:FEAT], wpad], axis=1)
    nwb = jnp.concatenate([node_w[FEAT:], wpad], axis=1)
    p, q = _tc_gcnfin2(hc, accg2, xs2, gcn_b2, nwa, nwb)

    acch2a = _sc_scatter(gw2, node_idx, he_idx, zeros128, HE_NNZ, N_EDGES)
    pg, qg = _sc_pq(p, q, row_idx, col_idx)
    he2 = _tc_hescale(hb, acch2a)

    acch2b = _sc_scatter(he2, he_idx, node_idx, zeros128, HE_NNZ, N_EDGES)
    nm, em = _tc_maps(hd, acch2b, hgc_b2, edge_w, edge_b, pg, qg, node_b)
    return _contrast(nm, em)


# speedup vs baseline: 1.0017x; 1.0017x over previous
import functools

import jax
import jax.numpy as jnp
from jax import lax
from jax.experimental import pallas as pl
from jax.experimental.pallas import tpu as pltpu
from jax.experimental.pallas import tpu_sc as plsc

N_NODES = 10000
N_NODES_PAD = 10112
N_EDGES = 8192
HE_NNZ = 32768
FEAT = 128
MAP = 64
LOG2 = 0.6931471805599453

NC = 2
NS = 16


def _leaky(x):
    return jnp.where(x >= 0, x, 0.01 * x)


def _sc_mesh():
    return plsc.VectorSubcoreMesh(core_axis_name="c", subcore_axis_name="s")


def _zero_table(zeros_hbm, table, sid, rows_per_tile):
    r0 = sid * rows_per_tile
    pltpu.sync_copy(zeros_hbm.at[pl.ds(r0, rows_per_tile)],
                    table.at[pl.ds(r0, rows_per_tile)])


def _writeback(table, out_hbm, sid, rows_per_tile):
    r0 = sid * rows_per_tile
    pltpu.sync_copy(table.at[pl.ds(r0, rows_per_tile)],
                    out_hbm.at[pl.ds(r0, rows_per_tile)])


def _ones_scatter_tile(idx_hbm, table, idx_v, ones_v, sid, n_items):
    per_tile = n_items // NS
    for c in range(per_tile // 256):
        off = sid * per_tile + c * 256
        pltpu.sync_copy(idx_hbm.at[pl.ds(off, 256)], idx_v)
        pltpu.sync_copy(ones_v, table.at[idx_v], add=True)


def _hist_body(col_hbm, node_hbm, he_hbm, ones_hbm, zeros_hbm,
               hc_out, hd_out, hb_out,
               table, idx_v, ones_v):
    cid = lax.axis_index("c")
    sid = lax.axis_index("s")
    pltpu.sync_copy(ones_hbm, ones_v)

    @pl.when(cid == 0)
    def _core0():
        _zero_table(zeros_hbm, table, sid, N_NODES_PAD // NS)
        plsc.subcore_barrier()
        _ones_scatter_tile(col_hbm, table, idx_v, ones_v, sid, N_EDGES)
        plsc.subcore_barrier()
        _writeback(table, hc_out, sid, N_NODES_PAD // NS)
        plsc.subcore_barrier()
        _zero_table(zeros_hbm, table, sid, N_EDGES // NS)
        plsc.subcore_barrier()
        _ones_scatter_tile(node_hbm, table, idx_v, ones_v, sid, HE_NNZ)
        plsc.subcore_barrier()
        _writeback(table, hd_out, sid, N_EDGES // NS)

    @pl.when(cid == 1)
    def _core1():
        _zero_table(zeros_hbm, table, sid, N_EDGES // NS)
        plsc.subcore_barrier()
        _ones_scatter_tile(he_hbm, table, idx_v, ones_v, sid, HE_NNZ)
        plsc.subcore_barrier()
        _writeback(table, hb_out, sid, N_EDGES // NS)


def _sc_hist(col_idx, node_idx, he_idx, ones128, zeros128):
    f32 = jnp.float32
    fn = pl.kernel(
        _hist_body,
        mesh=_sc_mesh(),
        out_type=[
            jax.ShapeDtypeStruct((N_NODES_PAD, FEAT), f32),
            jax.ShapeDtypeStruct((N_EDGES, FEAT), f32),
            jax.ShapeDtypeStruct((N_EDGES, FEAT), f32),
        ],
        scratch_types=[
            pltpu.VMEM_SHARED((N_NODES_PAD, FEAT), f32),
            pltpu.VMEM((256,), jnp.int32),
            pltpu.VMEM((256, FEAT), f32),
        ],
    )
    return fn(col_idx, node_idx, he_idx, ones128, zeros128)


def _partial_body(src_hbm, gidx_hbm, sidx_hbm, zeros_hbm, acc_out,
                  table, idx_v, rows_v, sem, *, n_items, n_rows):
    cid = lax.axis_index("c")
    sid = lax.axis_index("s")
    wid = sid * NC + cid
    rpt = n_rows // NS
    _zero_table(zeros_hbm, table, sid, rpt)
    plsc.subcore_barrier()
    per_w = n_items // (NC * NS)
    for c in range(per_w // 256):
        off = wid * per_w + c * 256
        pltpu.sync_copy(gidx_hbm.at[pl.ds(off, 256)], idx_v)
        pltpu.async_copy(src_hbm.at[idx_v], rows_v, sem).wait()
        pltpu.sync_copy(sidx_hbm.at[pl.ds(off, 256)], idx_v)
        pltpu.sync_copy(rows_v, table.at[idx_v], add=True)
    plsc.subcore_barrier()
    r0 = sid * rpt
    pltpu.sync_copy(table.at[pl.ds(r0, rpt)], acc_out.at[cid, pl.ds(r0, rpt)])


def _sc_scatter(src, gidx, sidx, zeros128, n_items, n_rows):
    f32 = jnp.float32
    body = functools.partial(_partial_body, n_items=n_items, n_rows=n_rows)
    fn = pl.kernel(
        body,
        mesh=_sc_mesh(),
        out_type=jax.ShapeDtypeStruct((NC, n_rows, FEAT), f32),
        scratch_types=[
            pltpu.VMEM_SHARED((n_rows, FEAT), f32),
            pltpu.VMEM((256,), jnp.int32),
            pltpu.VMEM((256, FEAT), f32),
            pltpu.SemaphoreType.DMA,
        ],
    )
    return fn(src, gidx, sidx, zeros128)


def _pq_body(p_hbm, q_hbm, row_hbm, col_hbm, pg_out, qg_out,
             idx_v, rows_v, sem):
    cid = lax.axis_index("c")
    sid = lax.axis_index("s")

    @pl.when(cid == 0)
    def _core0():
        for c in range(2):
            base = sid * 512 + c * 256
            pltpu.sync_copy(row_hbm.at[pl.ds(base, 256)], idx_v)
            pltpu.async_copy(p_hbm.at[idx_v], rows_v, sem).wait()
            pltpu.sync_copy(rows_v, pg_out.at[pl.ds(base, 256)])

    @pl.when(cid == 1)
    def _core1():
        for c in range(2):
            base = sid * 512 + c * 256
            pltpu.sync_copy(col_hbm.at[pl.ds(base, 256)], idx_v)
            pltpu.async_copy(q_hbm.at[idx_v], rows_v, sem).wait()
            pltpu.sync_copy(rows_v, qg_out.at[pl.ds(base, 256)])


def _sc_pq(p, q, row_idx, col_idx):
    f32 = jnp.float32
    fn = pl.kernel(
        _pq_body,
        mesh=_sc_mesh(),
        out_type=[
            jax.ShapeDtypeStruct((N_EDGES, FEAT), f32),
            jax.ShapeDtypeStruct((N_EDGES, FEAT), f32),
        ],
        scratch_types=[
            pltpu.VMEM((256,), jnp.int32),
            pltpu.VMEM((256, FEAT), f32),
            pltpu.SemaphoreType.DMA,
        ],
    )
    return fn(p, q, row_idx, col_idx)


def _dinv_from_hist(hc_ref):
    h = hc_ref[:, 0] + 1.0
    return (1.0 / jnp.sqrt(h))[:N_NODES, None]


def _recip_from_hist(hr_ref):
    h = hr_ref[:, 0]
    return jnp.where(h > 0, 1.0 / h, 0.0)[:, None]


def _mm1_body(nodes_ref, w1_ref, edges_ref, wh_ref, xw_ref, g_ref):
    xw_ref[...] = jnp.dot(nodes_ref[...], w1_ref[...],
                          preferred_element_type=jnp.float32)
    g_ref[...] = jnp.dot(edges_ref[...], wh_ref[...],
                         preferred_element_type=jnp.float32)


def _tc_mm1(nodes, w1, edges, wh):
    return pl.pallas_call(
        _mm1_body,
        out_shape=[jax.ShapeDtypeStruct((N_NODES, FEAT), jnp.float32),
                   jax.ShapeDtypeStruct((N_EDGES, FEAT), jnp.float32)],
    )(nodes, w1, edges, wh)


def _scale_body(hc_ref, xw_ref, xs_ref):
    xs_ref[...] = _dinv_from_hist(hc_ref) * xw_ref[...]


def _tc_scale(hc, xw):
    return pl.pallas_call(
        _scale_body,
        out_shape=jax.ShapeDtypeStruct((N_NODES, FEAT), jnp.float32),
    )(hc, xw)


def _gcnfin_body(hc_ref, accg_ref, xs_ref, b_ref, w_ref, out_ref):
    dinv = _dinv_from_hist(hc_ref)
    acc = accg_ref[0, :N_NODES, :] + accg_ref[1, :N_NODES, :]
    h = _leaky(dinv * (acc + xs_ref[...]) + b_ref[...])
    xw = jnp.dot(h, w_ref[...], preferred_element_type=jnp.float32)
    out_ref[...] = dinv * xw


def _tc_gcnfin(hc, accg, xs, b, w):
    return pl.pallas_call(
        _gcnfin_body,
        out_shape=jax.ShapeDtypeStruct((N_NODES, FEAT), jnp.float32),
    )(hc, accg, xs, b[None, :], w)


def _gcnfin2_body(hc_ref, accg_ref, xs2_ref, b2_ref, nwa_ref, nwb_ref,
                  p_ref, q_ref):
    dinv = _dinv_from_hist(hc_ref)
    acc = accg_ref[0, :N_NODES, :] + accg_ref[1, :N_NODES, :]
    ne = _leaky(dinv * (acc + xs2_ref[...]) + b2_ref[...])
    p_ref[...] = jnp.dot(ne, nwa_ref[...], preferred_element_type=jnp.float32)
    q_ref[...] = jnp.dot(ne, nwb_ref[...], preferred_element_type=jnp.float32)


def _tc_gcnfin2(hc, accg, xs2, b2, nwa, nwb):
    return pl.pallas_call(
        _gcnfin2_body,
        out_shape=[jax.ShapeDtypeStruct((N_NODES, FEAT), jnp.float32),
                   jax.ShapeDtypeStruct((N_NODES, FEAT), jnp.float32)],
    )(hc, accg, xs2, b2[None, :], nwa, nwb)


def _hypfin_body(hd_ref, acch_ref, hb1_ref, hw2_ref, gw2_ref):
    dinv_h = _recip_from_hist(hd_ref)
    g2 = _leaky(dinv_h * (acch_ref[0] + acch_ref[1]) + hb1_ref[...])
    gw2_ref[...] = jnp.dot(g2, hw2_ref[...], preferred_element_type=jnp.float32)


def _tc_hypfin(hd, acch, hb1, hw2):
    return pl.pallas_call(
        _hypfin_body,
        out_shape=jax.ShapeDtypeStruct((N_EDGES, FEAT), jnp.float32),
    )(hd, acch, hb1[None, :], hw2)


def _he_body(hb_ref, acch_ref, he_ref):
    he_ref[...] = _recip_from_hist(hb_ref) * (acch_ref[0] + acch_ref[1])


def _tc_hescale(hb, acch):
    return pl.pallas_call(
        _he_body,
        out_shape=jax.ShapeDtypeStruct((N_EDGES, FEAT), jnp.float32),
    )(hb, acch)


def _maps_body(hd_ref, acch_ref, hb2_ref, ew_ref, eb_ref,
               pg_ref, qg_ref, nb_ref, nm_ref, em_ref):
    dinv_h = _recip_from_hist(hd_ref)
    ee = _leaky(dinv_h * (acch_ref[0] + acch_ref[1]) + hb2_ref[...])
    emap = jnp.dot(ee, ew_ref[...], preferred_element_type=jnp.float32)
    emap = emap + eb_ref[...]
    nmap = (pg_ref[...] + qg_ref[...])[:, :MAP] + nb_ref[...]
    nm_ref[...] = nmap * lax.rsqrt(jnp.sum(nmap * nmap, axis=1,
                                           keepdims=True))
    em_ref[...] = emap * lax.rsqrt(jnp.sum(emap * emap, axis=1,
                                           keepdims=True))


def _tc_maps(hd, acch, hb2, ew, eb, pg, qg, nb):
    return pl.pallas_call(
        _maps_body,
        out_shape=[jax.ShapeDtypeStruct((N_EDGES, MAP), jnp.float32),
                   jax.ShapeDtypeStruct((N_EDGES, MAP), jnp.float32)],
    )(hd, acch, hb2[None, :], ew, eb[None, :], pg, qg, nb[None, :])


def _contrast_body(nm_ref, em_ref, out_ref, rs_ref, cs_ref, d_ref, *, bj, e):
    j = pl.program_id(0)
    nj = pl.num_programs(0)
    nm = nm_ref[...]
    em = em_ref[...]
    s = lax.dot_general(nm, em, (((1,), (1,)), ((), ())),
                        preferred_element_type=jnp.float32)
    z = jnp.exp(-jnp.abs(s))

    zparts = [z[:, k * 128:(k + 1) * 128] for k in range(bj // 128)]
    acc = zparts[0]
    for zp in zparts[1:]:
        acc = acc + zp

    @pl.when(j == 0)
    def _init():
        rs_ref[...] = acc

    @pl.when(j > 0)
    def _acc():
        rs_ref[...] += acc

    cs_ref[0, pl.ds(j * bj, bj)] = jnp.sum(z, axis=0)

    nm_blk = nm_ref[pl.ds(j * bj, bj), :]
    d_ref[0, pl.ds(j * bj, bj)] = jnp.sum(nm_blk * em, axis=1)

    @pl.when(j == nj - 1)
    def _fin():
        rowsum = jnp.sum(rs_ref[...], axis=1)
        out_ref[0, :] = (jnp.abs(d_ref[0, :]) - LOG2
                         + jnp.log(rowsum + cs_ref[0, :]))


def _contrast(nm, em, *, bj=512):
    e = nm.shape[0]
    nj = e // bj
    body = functools.partial(_contrast_body, bj=bj, e=e)
    out = pl.pallas_call(
        body,
        grid=(nj,),
        in_specs=[
            pl.BlockSpec((e, nm.shape[1]), lambda j: (0, 0)),
            pl.BlockSpec((bj, em.shape[1]), lambda j: (j, 0)),
        ],
        out_specs=pl.BlockSpec((1, e), lambda j: (0, 0)),
        out_shape=jax.ShapeDtypeStruct((1, e), jnp.float32),
        scratch_shapes=[
            pltpu.VMEM((e, 128), jnp.float32),
            pltpu.VMEM((1, e), jnp.float32),
            pltpu.VMEM((1, e), jnp.float32),
        ],
    )(nm, em)
    return out[0]


def kernel(nodes_feature, edges_feature, edge_index, hyperedge_index,
           gcn_w1, gcn_b1, gcn_w2, gcn_b2,
           hgc_w1, hgc_b1, hgc_w2, hgc_b2,
           node_w, node_b, edge_w, edge_b):
    f32 = jnp.float32
    row_idx = edge_index[0]
    col_idx = edge_index[1]
    node_idx = hyperedge_index[0]
    he_idx = hyperedge_index[1]

    ones128 = jnp.ones((256, FEAT), f32)
    zeros128 = jnp.zeros((N_NODES_PAD, FEAT), f32)

    hc, hd, hb = _sc_hist(col_idx, node_idx, he_idx, ones128, zeros128)
    xw1, g1 = _tc_mm1(nodes_feature, gcn_w1, edges_feature, hgc_w1)
    xs1 = _tc_scale(hc, xw1)

    acch1a = _sc_scatter(g1, node_idx, he_idx, zeros128, HE_NNZ, N_EDGES)
    accg1 = _sc_scatter(xs1, row_idx, col_idx, zeros128, N_EDGES, N_NODES_PAD)
    he1 = _tc_hescale(hb, acch1a)
    xs2 = _tc_gcnfin(hc, accg1, xs1, gcn_b1, gcn_w2)

    acch1b = _sc_scatter(he1, he_idx, node_idx, zeros128, HE_NNZ, N_EDGES)
    accg2 = _sc_scatter(xs2, row_idx, col_idx, zeros128, N_EDGES, N_NODES_PAD)
    gw2 = _tc_hypfin(hd, acch1b, hgc_b1, hgc_w2)
    wpad = jnp.zeros((FEAT, FEAT - MAP), f32)
    nwa = jnp.concatenate([node_w[:FEAT], wpad], axis=1)
    nwb = jnp.concatenate([node_w[FEAT:], wpad], axis=1)
    p, q = _tc_gcnfin2(hc, accg2, xs2, gcn_b2, nwa, nwb)

    acch2a = _sc_scatter(gw2, node_idx, he_idx, zeros128, HE_NNZ, N_EDGES)
    pg, qg = _sc_pq(p, q, row_idx, col_idx)
    he2 = _tc_hescale(hb, acch2a)

    acch2b = _sc_scatter(he2, he_idx, node_idx, zeros128, HE_NNZ, N_EDGES)
    nm, em = _tc_maps(hd, acch2b, hgc_b2, edge_w, edge_b, pg, qg, node_b)
    return _contrast(nm, em)
